# baseline (speedup 1.0000x reference)
import functools

import jax
import jax.numpy as jnp
from jax import lax
from jax.experimental import pallas as pl
from jax.experimental.pallas import tpu as pltpu
from jax.experimental.pallas import tpu_sc as plsc

NUM_ITEM = 1000000
DIM = 32
BATCH = 16384

_info = plsc.get_sparse_core_info()
_NC, _NS, _L = _info.num_cores, _info.num_subcores, _info.num_lanes
_NW = _NC * _NS
_BPW = BATCH // _NW
_CHUNK = 128
_NCHUNK = _BPW // _CHUNK
_GROUPS = _BPW // _L
_ROWW = 128
_NROW = NUM_ITEM * DIM // _ROWW
_NTC = NUM_ITEM // 128
_TAIL = NUM_ITEM - _NTC * 128
_TCPW = _NTC // _NW + 1

_mesh = plsc.VectorSubcoreMesh(core_axis_name="c", subcore_axis_name="s")
_params = pltpu.CompilerParams(needs_layout_passes=False)


@functools.partial(
    pl.kernel,
    mesh=_mesh,
    out_type=jax.ShapeDtypeStruct((_NROW, _ROWW), jnp.float32),
    scratch_types=[
        pltpu.VMEM((DIM, 128), jnp.float32),
        pltpu.VMEM((DIM, _TAIL), jnp.float32),
        pltpu.VMEM((32, 128), jnp.float32),
    ],
    compiler_params=_params,
)
def _sc_transpose(tt_hbm, out_hbm, tbuf, ttail, obuf):
    wid = lax.axis_index("s") * _NC + lax.axis_index("c")
    iota = lax.iota(jnp.int32, _L)

    def body(i, carry):
        tc = i * _NW + wid

        @pl.when(tc < _NTC)
        def _():
            pltpu.sync_copy(tt_hbm.at[:, pl.ds(tc * 128, 128)], tbuf)
            for j in range(DIM):
                for h in range(8):
                    vals = tbuf[j, pl.ds(h * _L, _L)]
                    word = (h * _L + iota) * DIM + j
                    plsc.store_scatter(
                        obuf,
                        [lax.shift_right_logical(word, 7), word & 127],
                        vals)
            pltpu.sync_copy(obuf, out_hbm.at[pl.ds(tc * 32, 32)])

        return carry

    lax.fori_loop(0, _TCPW, body, 0)

    @pl.when(wid == _NW - 1)
    def _():
        pltpu.sync_copy(tt_hbm.at[:, pl.ds(_NTC * 128, _TAIL)], ttail)
        for j in range(DIM):
            for h in range(_TAIL // _L):
                vals = ttail[j, pl.ds(h * _L, _L)]
                word = (h * _L + iota) * DIM + j
                plsc.store_scatter(
                    obuf,
                    [lax.shift_right_logical(word, 7), word & 127],
                    vals)
        pltpu.sync_copy(obuf.at[pl.ds(0, _TAIL * DIM // _ROWW)],
                        out_hbm.at[pl.ds(_NTC * 32, _TAIL * DIM // _ROWW)])


@functools.partial(
    pl.kernel,
    mesh=_mesh,
    out_type=jax.ShapeDtypeStruct((BATCH,), jnp.float32),
    scratch_types=[
        pltpu.VMEM((_NCHUNK, _CHUNK), jnp.int32),
        pltpu.VMEM((_NCHUNK, _CHUNK), jnp.int32),
        pltpu.VMEM((_BPW,), jnp.int32),
        pltpu.VMEM((_BPW, _ROWW), jnp.float32),
        pltpu.VMEM((DIM, _L), jnp.float32),
        pltpu.VMEM((_BPW,), jnp.float32),
        pltpu.SemaphoreType.DMA,
    ],
    compiler_params=_params,
)
def _sc_score(idx_hbm, userb_hbm, table_hbm, out_hbm,
              idx_v, row_v, off_v, rows_v, u_v, out_v, sem):
    wid = lax.axis_index("s") * _NC + lax.axis_index("c")
    pltpu.sync_copy(idx_hbm.at[pl.ds(wid * _NCHUNK, _NCHUNK)], idx_v)
    pltpu.sync_copy(userb_hbm, u_v)

    for j in range(_NCHUNK):
        for k in range(_CHUNK // _L):
            v = idx_v[j, pl.ds(k * _L, _L)]
            row_v[j, pl.ds(k * _L, _L)] = lax.shift_right_logical(v, 2)
            off_v[pl.ds(j * _CHUNK + k * _L, _L)] = (v & 3) * DIM

    copies = []
    for j in range(_NCHUNK):
        copies.append(pltpu.async_copy(
            table_hbm.at[row_v.at[j]],
            rows_v.at[pl.ds(j * _CHUNK, _CHUNK)],
            sem))
    for c in copies:
        c.wait()

    def body(g, carry):
        item_ids = g * _L + lax.iota(jnp.int32, _L)
        coloff = off_v[pl.ds(g * _L, _L)]
        acc = jnp.zeros((_L,), jnp.float32)
        for j in range(DIM):
            vals = plsc.load_gather(rows_v, [item_ids, coloff + j])
            acc = acc + vals * u_v[j]
        out_v[pl.ds(g * _L, _L)] = acc
        return carry

    lax.fori_loop(0, _GROUPS, body, 0)
    pltpu.sync_copy(out_v, out_hbm.at[pl.ds(wid * _BPW, _BPW)])


def kernel(item_idx, user_emb, item_emb):
    idx2 = item_idx.astype(jnp.int32).reshape(_NW * _NCHUNK, _CHUNK)
    userb = jnp.broadcast_to(user_emb.reshape(DIM, 1), (DIM, _L))
    table4 = _sc_transpose(item_emb.T)
    out = _sc_score(idx2, userb, table4)
    return out.reshape(1, BATCH)


# speedup vs baseline: 1.3826x; 1.3826x over previous
import functools

import jax
import jax.numpy as jnp
from jax import lax
from jax.experimental import pallas as pl
from jax.experimental.pallas import tpu as pltpu
from jax.experimental.pallas import tpu_sc as plsc

NUM_ITEM = 1000000
DIM = 32
BATCH = 16384

_info = plsc.get_sparse_core_info()
_NC, _NS, _L = _info.num_cores, _info.num_subcores, _info.num_lanes
_NW = _NC * _NS
_BPW = BATCH // _NW
_CHUNK = 128
_NCHUNK = _BPW // _CHUNK
_GROUPS = _BPW // _L
_ROWW = 128
_NROW = NUM_ITEM * DIM // _ROWW

_BLK = 512
_NBLK = NUM_ITEM // _BLK
_NFULL = 1952
_BPWT = _NFULL // _NW
_TAIL = NUM_ITEM - _NBLK * _BLK + _BLK - 448
_TAILSTART = NUM_ITEM - 64

_mesh = plsc.VectorSubcoreMesh(core_axis_name="c", subcore_axis_name="s")
_params = pltpu.CompilerParams(needs_layout_passes=False)


@functools.partial(
    pl.kernel,
    mesh=_mesh,
    out_type=jax.ShapeDtypeStruct((_NROW, _ROWW), jnp.float32),
    scratch_types=[
        pltpu.VMEM((DIM, _BLK), jnp.float32),
        pltpu.VMEM((DIM, _BLK), jnp.float32),
        pltpu.VMEM((_BLK * DIM // _ROWW, _ROWW), jnp.float32),
        pltpu.VMEM((_BLK * DIM // _ROWW, _ROWW), jnp.float32),
        pltpu.VMEM((DIM, 64), jnp.float32),
        pltpu.SemaphoreType.DMA,
        pltpu.SemaphoreType.DMA,
        pltpu.SemaphoreType.DMA,
        pltpu.SemaphoreType.DMA,
    ],
    compiler_params=_params,
)
def _sc_transpose(tt_hbm, out_hbm, tbuf0, tbuf1, obuf0, obuf1, ttail,
                  sin0, sin1, sout0, sout1):
    wid = lax.axis_index("s") * _NC + lax.axis_index("c")
    iota = lax.iota(jnp.int32, _L)
    rbase = lax.shift_right_logical(iota, 2)
    cbase = (iota & 3) * DIM
    tbufs = (tbuf0, tbuf1)
    obufs = (obuf0, obuf1)
    sins = (sin0, sin1)
    souts = (sout0, sout1)
    blk0 = wid * _BPWT

    def cin(s, blk):
        return pltpu.make_async_copy(
            tt_hbm.at[:, pl.ds(blk * _BLK, _BLK)], tbufs[s], sins[s])

    def cout(s, blk):
        return pltpu.make_async_copy(
            obufs[s], out_hbm.at[pl.ds(blk * (_BLK * DIM // _ROWW),
                                       _BLK * DIM // _ROWW)], souts[s])

    def compute(s):
        tb, ob = tbufs[s], obufs[s]

        def hbody(h, carry):
            rows = h * 4 + rbase
            for j in range(DIM):
                vals = tb[j, pl.ds(h * _L, _L)]
                plsc.store_scatter(ob, [rows, cbase + j], vals)
            return carry

        lax.fori_loop(0, _BLK // _L, hbody, 0)

    cin(0, blk0).start()

    def pair(k, carry):
        for s in (0, 1):
            i = 2 * k + s
            blk = blk0 + i

            @pl.when(i < _BPWT - 1)
            def _():
                cin(1 - s, blk + 1).start()

            cin(s, blk).wait()

            @pl.when(i >= 2)
            def _():
                cout(s, blk - 2).wait()

            compute(s)
            cout(s, blk).start()
        return carry

    lax.fori_loop(0, (_BPWT - 1) // 2, pair, 0)

    lastblk = blk0 + _BPWT - 1
    cin(0, lastblk).wait()
    cout(0, lastblk - 2).wait()
    compute(0)
    cout(0, lastblk).start()
    cout(1, lastblk - 1).wait()
    cout(0, lastblk).wait()

    @pl.when(wid == _NW - 1)
    def _():
        cin(0, _NFULL).start()
        cin(0, _NFULL).wait()
        compute(0)
        cout(0, _NFULL).start()
        cout(0, _NFULL).wait()

        pltpu.sync_copy(tt_hbm.at[:, pl.ds(_TAILSTART, 64)], ttail)
        for h in range(64 // _L):
            rows = h * 4 + rbase
            for j in range(DIM):
                vals = ttail[j, pl.ds(h * _L, _L)]
                plsc.store_scatter(obuf0, [rows, cbase + j], vals)
        pltpu.sync_copy(obuf0.at[pl.ds(0, 64 * DIM // _ROWW)],
                        out_hbm.at[pl.ds(_TAILSTART * DIM // _ROWW,
                                         64 * DIM // _ROWW)])


@functools.partial(
    pl.kernel,
    mesh=_mesh,
    out_type=jax.ShapeDtypeStruct((BATCH,), jnp.float32),
    scratch_types=[
        pltpu.VMEM((_NCHUNK, _CHUNK), jnp.int32),
        pltpu.VMEM((_NCHUNK, _CHUNK), jnp.int32),
        pltpu.VMEM((_BPW,), jnp.int32),
        pltpu.VMEM((_BPW, _ROWW), jnp.float32),
        pltpu.VMEM((DIM, _L), jnp.float32),
        pltpu.VMEM((_BPW,), jnp.float32),
        pltpu.SemaphoreType.DMA,
    ],
    compiler_params=_params,
)
def _sc_score(idx_hbm, userb_hbm, table_hbm, out_hbm,
              idx_v, row_v, off_v, rows_v, u_v, out_v, sem):
    wid = lax.axis_index("s") * _NC + lax.axis_index("c")
    pltpu.sync_copy(idx_hbm.at[pl.ds(wid * _NCHUNK, _NCHUNK)], idx_v)
    pltpu.sync_copy(userb_hbm, u_v)

    for j in range(_NCHUNK):
        for k in range(_CHUNK // _L):
            v = idx_v[j, pl.ds(k * _L, _L)]
            row_v[j, pl.ds(k * _L, _L)] = lax.shift_right_logical(v, 2)
            off_v[pl.ds(j * _CHUNK + k * _L, _L)] = (v & 3) * DIM

    copies = []
    for j in range(_NCHUNK):
        copies.append(pltpu.async_copy(
            table_hbm.at[row_v.at[j]],
            rows_v.at[pl.ds(j * _CHUNK, _CHUNK)],
            sem))
    for c in copies:
        c.wait()

    def body(g, carry):
        item_ids = g * _L + lax.iota(jnp.int32, _L)
        coloff = off_v[pl.ds(g * _L, _L)]
        acc = jnp.zeros((_L,), jnp.float32)
        for j in range(DIM):
            vals = plsc.load_gather(rows_v, [item_ids, coloff + j])
            acc = acc + vals * u_v[j]
        out_v[pl.ds(g * _L, _L)] = acc
        return carry

    lax.fori_loop(0, _GROUPS, body, 0)
    pltpu.sync_copy(out_v, out_hbm.at[pl.ds(wid * _BPW, _BPW)])


def kernel(item_idx, user_emb, item_emb):
    idx2 = item_idx.astype(jnp.int32).reshape(_NW * _NCHUNK, _CHUNK)
    userb = jnp.broadcast_to(user_emb.reshape(DIM, 1), (DIM, _L))
    table4 = _sc_transpose(item_emb.T)
    out = _sc_score(idx2, userb, table4)
    return out.reshape(1, BATCH)
